# Initial kernel scaffold; baseline (speedup 1.0000x reference)
import functools

import jax
import jax.numpy as jnp
from jax import lax
from jax.experimental import pallas as pl
from jax.experimental.pallas import tpu as pltpu
from jax.experimental.pallas import tpu_sc as plsc

N = 10000
E = 320000
D = 128
H = 128
NE = 64

NPAD = 10240
HA = 144
NC = 2
NS = 16
NW = NC * NS
CH = 128
NCH = 80
EPAD = NW * NCH * CH
JUNK = N + 100
ROWS_PER_TEC = NPAD // NS
TC_BLK = 512
TC_GRID = NPAD // TC_BLK


def _leaky(x):
    return jnp.where(x > 0, x, 0.2 * x)


def _prep_tail(h, a_src_ref, a_dst_ref, haug_ref, asv_ref, adv_ref, wself_ref):
    asv = jax.lax.dot_general(h, a_src_ref[...], (((1,), (0,)), ((), ())),
                              precision=lax.Precision.HIGHEST,
                              preferred_element_type=jnp.float32)
    adv = jax.lax.dot_general(h, a_dst_ref[...], (((1,), (0,)), ((), ())),
                              precision=lax.Precision.HIGHEST,
                              preferred_element_type=jnp.float32)
    wself = jnp.exp(_leaky(asv + adv))
    ones_col = jnp.where(
        lax.broadcasted_iota(jnp.int32, (TC_BLK, HA - H), 1) == 0, 1.0, 0.0)
    haug_ref[...] = jnp.concatenate([h, ones_col], axis=1)
    asv_ref[...] = asv
    adv_ref[...] = adv
    wself_ref[...] = wself


def _tc_prep_kernel(x_ref, w_ref, a_src_ref, a_dst_ref,
                    haug_ref, asv_ref, adv_ref, wself_ref):
    h = jax.lax.dot_general(x_ref[...], w_ref[...], (((1,), (0,)), ((), ())),
                            precision=lax.Precision.HIGHEST,
                            preferred_element_type=jnp.float32)
    _prep_tail(h, a_src_ref, a_dst_ref, haug_ref, asv_ref, adv_ref, wself_ref)


def _epilogue(acc_ref, haug_ref, wself_ref, b_ref):
    a = acc_ref[0] + acc_ref[1]
    hprev = haug_ref[...][:, :H]
    wself = wself_ref[...]
    numer = a[:, :H] + wself * hprev
    den = jnp.sum(a[:, H:], axis=1, keepdims=True) + wself + 1e-16
    hid = jnp.maximum(numer / den + b_ref[...], 0.0)
    i = pl.program_id(0)
    rows = i * TC_BLK + lax.broadcasted_iota(jnp.int32, (TC_BLK, 1), 0)
    return jnp.where(rows < N, hid, 0.0)


def _tc_mid_kernel(acc_ref, haug_ref, wself_ref, b_ref, w_ref, a_src_ref,
                   a_dst_ref, haug2_ref, asv_ref, adv_ref, wself2_ref):
    hid = _epilogue(acc_ref, haug_ref, wself_ref, b_ref)
    h2 = jax.lax.dot_general(hid, w_ref[...], (((1,), (0,)), ((), ())),
                             precision=lax.Precision.HIGHEST,
                             preferred_element_type=jnp.float32)
    _prep_tail(h2, a_src_ref, a_dst_ref, haug2_ref, asv_ref, adv_ref, wself2_ref)


def _tc_fin_kernel(acc_ref, haug_ref, wself_ref, b_ref, wl_ref, bl_ref, out_ref):
    hid = _epilogue(acc_ref, haug_ref, wself_ref, b_ref)
    out_ref[...] = jax.lax.dot_general(
        hid, wl_ref[...], (((1,), (0,)), ((), ())),
        precision=lax.Precision.HIGHEST,
        preferred_element_type=jnp.float32) + bl_ref[...]


def _row_spec(width):
    return pl.BlockSpec((TC_BLK, width), lambda i: (i, 0))


def _whole(shape):
    return pl.BlockSpec(shape, lambda i: tuple(0 for _ in shape))


def _tc_prep(x, w, a_src, a_dst):
    return pl.pallas_call(
        _tc_prep_kernel,
        grid=(TC_GRID,),
        in_specs=[_row_spec(D), _whole((D, H)), _whole((H, 1)), _whole((H, 1))],
        out_specs=[_row_spec(HA), _row_spec(1), _row_spec(1), _row_spec(1)],
        out_shape=[
            jax.ShapeDtypeStruct((NPAD, HA), jnp.float32),
            jax.ShapeDtypeStruct((NPAD, 1), jnp.float32),
            jax.ShapeDtypeStruct((NPAD, 1), jnp.float32),
            jax.ShapeDtypeStruct((NPAD, 1), jnp.float32),
        ],
    )(x, w, a_src, a_dst)


def _tc_mid(acc, haug, wself, b, w, a_src, a_dst):
    return pl.pallas_call(
        _tc_mid_kernel,
        grid=(TC_GRID,),
        in_specs=[
            pl.BlockSpec((NC, TC_BLK, HA), lambda i: (0, i, 0)),
            _row_spec(HA), _row_spec(1), _whole((1, H)),
            _whole((H, H)), _whole((H, 1)), _whole((H, 1)),
        ],
        out_specs=[_row_spec(HA), _row_spec(1), _row_spec(1), _row_spec(1)],
        out_shape=[
            jax.ShapeDtypeStruct((NPAD, HA), jnp.float32),
            jax.ShapeDtypeStruct((NPAD, 1), jnp.float32),
            jax.ShapeDtypeStruct((NPAD, 1), jnp.float32),
            jax.ShapeDtypeStruct((NPAD, 1), jnp.float32),
        ],
    )(acc, haug, wself, b, w, a_src, a_dst)


def _tc_fin(acc, haug, wself, b, wl, bl):
    return pl.pallas_call(
        _tc_fin_kernel,
        grid=(TC_GRID,),
        in_specs=[
            pl.BlockSpec((NC, TC_BLK, HA), lambda i: (0, i, 0)),
            _row_spec(HA), _row_spec(1), _whole((1, H)),
            _whole((H, NE)), _whole((1, NE)),
        ],
        out_specs=_row_spec(NE),
        out_shape=jax.ShapeDtypeStruct((NPAD, NE), jnp.float32),
    )(acc, haug, wself, b, wl, bl)


def _sc_gat_body(haug, asv, adv, srcs, dsts, out,
                 src_s, dst_s, as_t, ad_t, w_b, rbuf, acc, gsem):
    c = lax.axis_index("c")
    s = lax.axis_index("s")
    wid = c * NS + s

    pltpu.sync_copy(srcs.at[wid], src_s)
    pltpu.sync_copy(dsts.at[wid], dst_s)
    pltpu.sync_copy(asv, as_t)
    pltpu.sync_copy(adv, ad_t)

    zero = jnp.zeros((16,), jnp.float32)

    def zrow(r, carry):
        for q in range(HA // 16):
            rbuf[r, pl.ds(q * 16, 16)] = zero
        return carry

    lax.fori_loop(0, CH, zrow, 0)
    for k in range(ROWS_PER_TEC // CH):
        pltpu.sync_copy(rbuf, acc.at[pl.ds(s * ROWS_PER_TEC + k * CH, CH)])
    plsc.subcore_barrier()

    def chunk_body(j, carry):
        cp = pltpu.async_copy(haug.at[src_s.at[j]], rbuf, gsem)
        for v in range(CH // 16):
            sv = src_s[j, pl.ds(v * 16, 16)]
            dv = dst_s[j, pl.ds(v * 16, 16)]
            t = plsc.load_gather(as_t, [sv]) + plsc.load_gather(ad_t, [dv])
            w_b[pl.ds(v * 16, 16)] = jnp.exp(jnp.where(t > 0, t, 0.2 * t))
        cp.wait()

        def srow(r, carry2):
            wv = plsc.load_gather(w_b, [jnp.zeros((16,), jnp.int32) + r])
            for q in range(HA // 16):
                rbuf[r, pl.ds(q * 16, 16)] = rbuf[r, pl.ds(q * 16, 16)] * wv
            return carry2

        lax.fori_loop(0, CH, srow, 0)
        pltpu.sync_copy(rbuf, acc.at[dst_s.at[j]], add=True)
        return carry

    lax.fori_loop(0, NCH, chunk_body, 0)
    plsc.subcore_barrier()
    pltpu.sync_copy(acc.at[pl.ds(s * ROWS_PER_TEC, ROWS_PER_TEC)],
                    out.at[c, pl.ds(s * ROWS_PER_TEC, ROWS_PER_TEC)])


_sc_gat = functools.partial(
    pl.kernel,
    out_type=jax.ShapeDtypeStruct((NC, NPAD, HA), jnp.float32),
    mesh=plsc.VectorSubcoreMesh(core_axis_name="c", subcore_axis_name="s"),
    scratch_types=[
        pltpu.VMEM((NCH, CH), jnp.int32),
        pltpu.VMEM((NCH, CH), jnp.int32),
        pltpu.VMEM((NPAD,), jnp.float32),
        pltpu.VMEM((NPAD,), jnp.float32),
        pltpu.VMEM((CH,), jnp.float32),
        pltpu.VMEM((CH, HA), jnp.float32),
        pltpu.VMEM_SHARED((NPAD, HA), jnp.float32),
        pltpu.SemaphoreType.DMA,
    ],
)(_sc_gat_body)


def kernel(x, edge_index, W1, a_src1, a_dst1, b1, W2, a_src2, a_dst2, b2, Wl, bl):
    src = edge_index[0]
    dst = edge_index[1]
    pad = EPAD - E
    srcs = jnp.concatenate(
        [src, jnp.zeros((pad,), jnp.int32)]).reshape(NW, NCH, CH)
    dsts = jnp.concatenate(
        [dst, jnp.full((pad,), JUNK, jnp.int32)]).reshape(NW, NCH, CH)

    xp = jnp.concatenate([x, jnp.zeros((NPAD - N, D), jnp.float32)])

    a_src1c = a_src1.reshape(H, 1)
    a_dst1c = a_dst1.reshape(H, 1)
    a_src2c = a_src2.reshape(H, 1)
    a_dst2c = a_dst2.reshape(H, 1)

    haug1, asv1, adv1, wself1 = _tc_prep(xp, W1, a_src1c, a_dst1c)
    acc1 = _sc_gat(haug1, asv1.reshape(NPAD), adv1.reshape(NPAD), srcs, dsts)
    haug2, asv2, adv2, wself2 = _tc_mid(
        acc1, haug1, wself1, b1.reshape(1, H), W2, a_src2c, a_dst2c)
    acc2 = _sc_gat(haug2, asv2.reshape(NPAD), adv2.reshape(NPAD), srcs, dsts)
    out = _tc_fin(acc2, haug2, wself2, b2.reshape(1, H), Wl, bl.reshape(1, NE))

    return out[:N].reshape(-1, 2000 * NE)


# speedup vs baseline: 15.7635x; 15.7635x over previous
import functools

import jax
import jax.numpy as jnp
from jax import lax
from jax.experimental import pallas as pl
from jax.experimental.pallas import tpu as pltpu
from jax.experimental.pallas import tpu_sc as plsc

N = 10000
E = 320000
D = 128
H = 128
NE = 64

NPAD = 10240
HA = 144
NC = 2
NS = 16
NW = NC * NS
CH = 128
NCH = 80
EPAD = NW * NCH * CH
ACC_ROWS = 10016
JUNK = N + 8
ROWS_PER_TEC = ACC_ROWS // NS
TC_BLK = 512
TC_GRID = NPAD // TC_BLK


def _leaky(x):
    return jnp.where(x > 0, x, 0.2 * x)


def _prep_tail(h, a_src_ref, a_dst_ref, haug_ref, asv_ref, adv_ref, wself_ref):
    asv = jax.lax.dot_general(h, a_src_ref[...], (((1,), (0,)), ((), ())),
                              precision=lax.Precision.HIGHEST,
                              preferred_element_type=jnp.float32)
    adv = jax.lax.dot_general(h, a_dst_ref[...], (((1,), (0,)), ((), ())),
                              precision=lax.Precision.HIGHEST,
                              preferred_element_type=jnp.float32)
    wself = jnp.exp(_leaky(asv + adv))
    ones_col = jnp.where(
        lax.broadcasted_iota(jnp.int32, (TC_BLK, HA - H), 1) == 0, 1.0, 0.0)
    haug_ref[...] = jnp.concatenate([h, ones_col], axis=1)
    asv_ref[...] = asv
    adv_ref[...] = adv
    wself_ref[...] = wself


def _tc_prep_kernel(x_ref, w_ref, a_src_ref, a_dst_ref,
                    haug_ref, asv_ref, adv_ref, wself_ref):
    h = jax.lax.dot_general(x_ref[...], w_ref[...], (((1,), (0,)), ((), ())),
                            precision=lax.Precision.HIGHEST,
                            preferred_element_type=jnp.float32)
    _prep_tail(h, a_src_ref, a_dst_ref, haug_ref, asv_ref, adv_ref, wself_ref)


def _epilogue(acc_ref, haug_ref, wself_ref, b_ref):
    a = acc_ref[0] + acc_ref[1]
    hprev = haug_ref[...][:, :H]
    wself = wself_ref[...]
    numer = a[:, :H] + wself * hprev
    den = jnp.sum(a[:, H:], axis=1, keepdims=True) + wself + 1e-16
    hid = jnp.maximum(numer / den + b_ref[...], 0.0)
    i = pl.program_id(0)
    rows = i * TC_BLK + lax.broadcasted_iota(jnp.int32, (TC_BLK, 1), 0)
    return jnp.where(rows < N, hid, 0.0)


def _tc_mid_kernel(acc_ref, haug_ref, wself_ref, b_ref, w_ref, a_src_ref,
                   a_dst_ref, haug2_ref, asv_ref, adv_ref, wself2_ref):
    hid = _epilogue(acc_ref, haug_ref, wself_ref, b_ref)
    h2 = jax.lax.dot_general(hid, w_ref[...], (((1,), (0,)), ((), ())),
                             precision=lax.Precision.HIGHEST,
                             preferred_element_type=jnp.float32)
    _prep_tail(h2, a_src_ref, a_dst_ref, haug2_ref, asv_ref, adv_ref, wself2_ref)


def _tc_fin_kernel(acc_ref, haug_ref, wself_ref, b_ref, wl_ref, bl_ref, out_ref):
    hid = _epilogue(acc_ref, haug_ref, wself_ref, b_ref)
    out_ref[...] = jax.lax.dot_general(
        hid, wl_ref[...], (((1,), (0,)), ((), ())),
        precision=lax.Precision.HIGHEST,
        preferred_element_type=jnp.float32) + bl_ref[...]


def _row_spec(width):
    return pl.BlockSpec((TC_BLK, width), lambda i: (i, 0))


def _whole(shape):
    return pl.BlockSpec(shape, lambda i: tuple(0 for _ in shape))


def _tc_prep(x, w, a_src, a_dst):
    return pl.pallas_call(
        _tc_prep_kernel,
        grid=(TC_GRID,),
        in_specs=[_row_spec(D), _whole((D, H)), _whole((H, 1)), _whole((H, 1))],
        out_specs=[_row_spec(HA), _row_spec(1), _row_spec(1), _row_spec(1)],
        out_shape=[
            jax.ShapeDtypeStruct((NPAD, HA), jnp.float32),
            jax.ShapeDtypeStruct((NPAD, 1), jnp.float32),
            jax.ShapeDtypeStruct((NPAD, 1), jnp.float32),
            jax.ShapeDtypeStruct((NPAD, 1), jnp.float32),
        ],
    )(x, w, a_src, a_dst)


def _tc_mid(acc, haug, wself, b, w, a_src, a_dst):
    return pl.pallas_call(
        _tc_mid_kernel,
        grid=(TC_GRID,),
        in_specs=[
            pl.BlockSpec((NC, TC_BLK, HA), lambda i: (0, i, 0)),
            _row_spec(HA), _row_spec(1), _whole((1, H)),
            _whole((H, H)), _whole((H, 1)), _whole((H, 1)),
        ],
        out_specs=[_row_spec(HA), _row_spec(1), _row_spec(1), _row_spec(1)],
        out_shape=[
            jax.ShapeDtypeStruct((NPAD, HA), jnp.float32),
            jax.ShapeDtypeStruct((NPAD, 1), jnp.float32),
            jax.ShapeDtypeStruct((NPAD, 1), jnp.float32),
            jax.ShapeDtypeStruct((NPAD, 1), jnp.float32),
        ],
    )(acc, haug, wself, b, w, a_src, a_dst)


def _tc_fin(acc, haug, wself, b, wl, bl):
    return pl.pallas_call(
        _tc_fin_kernel,
        grid=(TC_GRID,),
        in_specs=[
            pl.BlockSpec((NC, TC_BLK, HA), lambda i: (0, i, 0)),
            _row_spec(HA), _row_spec(1), _whole((1, H)),
            _whole((H, NE)), _whole((1, NE)),
        ],
        out_specs=_row_spec(NE),
        out_shape=jax.ShapeDtypeStruct((NPAD, NE), jnp.float32),
    )(acc, haug, wself, b, wl, bl)


def _sc_gat_body(haug, asv, adv, eidx, out,
                 ebuf, as_t, ad_t, w_b, rbuf, acc, gsem, isem):
    c = lax.axis_index("c")
    s = lax.axis_index("s")
    wid = c * NS + s

    pltpu.sync_copy(asv.at[pl.ds(0, ACC_ROWS)], as_t)
    pltpu.sync_copy(adv.at[pl.ds(0, ACC_ROWS)], ad_t)

    zero = jnp.zeros((16,), jnp.float32)

    def zrow(r, carry):
        for q in range(HA // 16):
            rbuf[r, pl.ds(q * 16, 16)] = zero
        return carry

    lax.fori_loop(0, CH, zrow, 0)
    base = s * ROWS_PER_TEC
    for k in range(ROWS_PER_TEC // CH):
        pltpu.sync_copy(rbuf, acc.at[pl.ds(base + k * CH, CH)])
    rem = ROWS_PER_TEC % CH
    if rem:
        pltpu.sync_copy(rbuf.at[pl.ds(0, rem)],
                        acc.at[pl.ds(base + (ROWS_PER_TEC // CH) * CH, rem)])
    plsc.subcore_barrier()

    pltpu.async_copy(eidx.at[wid, 0], ebuf.at[0], isem)

    def chunk_body(j, carry):
        b = j % 2
        pltpu.make_async_copy(eidx.at[wid, j], ebuf.at[b], isem).wait()

        @pl.when(j + 1 < NCH)
        def _():
            pltpu.async_copy(eidx.at[wid, j + 1], ebuf.at[1 - b], isem)

        cp = pltpu.async_copy(haug.at[ebuf.at[b, 0]], rbuf, gsem)
        for v in range(CH // 16):
            sv = ebuf[b, 0, pl.ds(v * 16, 16)]
            dv = ebuf[b, 1, pl.ds(v * 16, 16)]
            t = plsc.load_gather(as_t, [sv]) + plsc.load_gather(ad_t, [dv])
            w_b[pl.ds(v * 16, 16)] = jnp.exp(jnp.where(t > 0, t, 0.2 * t))
        cp.wait()

        def srow(r, carry2):
            wv = plsc.load_gather(w_b, [jnp.zeros((16,), jnp.int32) + r])
            for q in range(HA // 16):
                rbuf[r, pl.ds(q * 16, 16)] = rbuf[r, pl.ds(q * 16, 16)] * wv
            return carry2

        lax.fori_loop(0, CH, srow, 0)
        pltpu.sync_copy(rbuf, acc.at[ebuf.at[b, 1]], add=True)
        return carry

    lax.fori_loop(0, NCH, chunk_body, 0)
    plsc.subcore_barrier()
    pltpu.sync_copy(acc.at[pl.ds(s * ROWS_PER_TEC, ROWS_PER_TEC)],
                    out.at[c, pl.ds(s * ROWS_PER_TEC, ROWS_PER_TEC)])


_sc_gat = functools.partial(
    pl.kernel,
    out_type=jax.ShapeDtypeStruct((NC, NPAD, HA), jnp.float32),
    mesh=plsc.VectorSubcoreMesh(core_axis_name="c", subcore_axis_name="s"),
    compiler_params=pltpu.CompilerParams(
        needs_layout_passes=False, use_tc_tiling_on_sc=False),
    scratch_types=[
        pltpu.VMEM((2, 2, CH), jnp.int32),
        pltpu.VMEM((ACC_ROWS,), jnp.float32),
        pltpu.VMEM((ACC_ROWS,), jnp.float32),
        pltpu.VMEM((CH,), jnp.float32),
        pltpu.VMEM((CH, HA), jnp.float32),
        pltpu.VMEM_SHARED((ACC_ROWS, HA), jnp.float32),
        pltpu.SemaphoreType.DMA,
        pltpu.SemaphoreType.DMA,
    ],
)(_sc_gat_body)


def kernel(x, edge_index, W1, a_src1, a_dst1, b1, W2, a_src2, a_dst2, b2, Wl, bl):
    src = edge_index[0]
    dst = edge_index[1]
    pad = EPAD - E
    srcs = jnp.concatenate(
        [src, jnp.zeros((pad,), jnp.int32)]).reshape(NW, NCH, CH)
    dsts = jnp.concatenate(
        [dst, jnp.full((pad,), JUNK, jnp.int32)]).reshape(NW, NCH, CH)
    eidx = jnp.stack([srcs, dsts], axis=2)

    xp = jnp.concatenate([x, jnp.zeros((NPAD - N, D), jnp.float32)])

    a_src1c = a_src1.reshape(H, 1)
    a_dst1c = a_dst1.reshape(H, 1)
    a_src2c = a_src2.reshape(H, 1)
    a_dst2c = a_dst2.reshape(H, 1)

    haug1, asv1, adv1, wself1 = _tc_prep(xp, W1, a_src1c, a_dst1c)
    acc1 = _sc_gat(haug1, asv1.reshape(NPAD), adv1.reshape(NPAD), eidx)
    haug2, asv2, adv2, wself2 = _tc_mid(
        acc1, haug1, wself1, b1.reshape(1, H), W2, a_src2c, a_dst2c)
    acc2 = _sc_gat(haug2, asv2.reshape(NPAD), adv2.reshape(NPAD), eidx)
    out = _tc_fin(acc2, haug2, wself2, b2.reshape(1, H), Wl, bl.reshape(1, NE))

    return out[:N].reshape(-1, 2000 * NE)


# speedup vs baseline: 21.8344x; 1.3851x over previous
import functools

import jax
import jax.numpy as jnp
from jax import lax
from jax.experimental import pallas as pl
from jax.experimental.pallas import tpu as pltpu
from jax.experimental.pallas import tpu_sc as plsc

N = 10000
E = 320000
D = 128
H = 128
NE = 64

NPAD = 10240
HA = 144
NC = 2
NS = 16
NW = NC * NS
CH = 96
NCH = 106
EPAD = NW * NCH * CH
ACC_ROWS = 10016
JUNK = N + 8
ROWS_PER_TEC = ACC_ROWS // NS
TC_BLK = 512
TC_GRID = NPAD // TC_BLK


def _leaky(x):
    return jnp.where(x > 0, x, 0.2 * x)


def _prep_tail(h, a_src_ref, a_dst_ref, haug_ref, adv_ref, wself_ref):
    asv = jax.lax.dot_general(h, a_src_ref[...], (((1,), (0,)), ((), ())),
                              precision=lax.Precision.HIGHEST,
                              preferred_element_type=jnp.float32)
    adv = jax.lax.dot_general(h, a_dst_ref[...], (((1,), (0,)), ((), ())),
                              precision=lax.Precision.HIGHEST,
                              preferred_element_type=jnp.float32)
    wself = jnp.exp(_leaky(asv + adv))
    tail_iota = lax.broadcasted_iota(jnp.int32, (TC_BLK, HA - H), 1)
    extra = jnp.where(tail_iota == 0, 1.0, jnp.where(tail_iota == 1, asv, 0.0))
    haug_ref[...] = jnp.concatenate([h, extra], axis=1)
    adv_ref[...] = adv
    wself_ref[...] = wself


def _tc_prep_kernel(x_ref, w_ref, a_src_ref, a_dst_ref,
                    haug_ref, adv_ref, wself_ref):
    h = jax.lax.dot_general(x_ref[...], w_ref[...], (((1,), (0,)), ((), ())),
                            precision=lax.Precision.HIGHEST,
                            preferred_element_type=jnp.float32)
    _prep_tail(h, a_src_ref, a_dst_ref, haug_ref, adv_ref, wself_ref)


def _epilogue(acc_ref, haug_ref, wself_ref, b_ref):
    a = acc_ref[0] + acc_ref[1]
    hprev = haug_ref[...][:, :H]
    wself = wself_ref[...]
    numer = a[:, :H] + wself * hprev
    tail_iota = lax.broadcasted_iota(jnp.int32, (TC_BLK, HA - H), 1)
    den = jnp.sum(jnp.where(tail_iota == 0, a[:, H:], 0.0),
                  axis=1, keepdims=True) + wself + 1e-16
    hid = jnp.maximum(numer / den + b_ref[...], 0.0)
    i = pl.program_id(0)
    rows = i * TC_BLK + lax.broadcasted_iota(jnp.int32, (TC_BLK, 1), 0)
    return jnp.where(rows < N, hid, 0.0)


def _tc_mid_kernel(acc_ref, haug_ref, wself_ref, b_ref, w_ref, a_src_ref,
                   a_dst_ref, haug2_ref, adv_ref, wself2_ref):
    hid = _epilogue(acc_ref, haug_ref, wself_ref, b_ref)
    h2 = jax.lax.dot_general(hid, w_ref[...], (((1,), (0,)), ((), ())),
                             precision=lax.Precision.HIGHEST,
                             preferred_element_type=jnp.float32)
    _prep_tail(h2, a_src_ref, a_dst_ref, haug2_ref, adv_ref, wself2_ref)


def _tc_fin_kernel(acc_ref, haug_ref, wself_ref, b_ref, wl_ref, bl_ref, out_ref):
    hid = _epilogue(acc_ref, haug_ref, wself_ref, b_ref)
    out_ref[...] = jax.lax.dot_general(
        hid, wl_ref[...], (((1,), (0,)), ((), ())),
        precision=lax.Precision.HIGHEST,
        preferred_element_type=jnp.float32) + bl_ref[...]


def _row_spec(width):
    return pl.BlockSpec((TC_BLK, width), lambda i: (i, 0))


def _whole(shape):
    return pl.BlockSpec(shape, lambda i: tuple(0 for _ in shape))


def _tc_prep(x, w, a_src, a_dst):
    return pl.pallas_call(
        _tc_prep_kernel,
        grid=(TC_GRID,),
        in_specs=[_row_spec(D), _whole((D, H)), _whole((H, 1)), _whole((H, 1))],
        out_specs=[_row_spec(HA), _row_spec(1), _row_spec(1)],
        out_shape=[
            jax.ShapeDtypeStruct((NPAD, HA), jnp.float32),
            jax.ShapeDtypeStruct((NPAD, 1), jnp.float32),
            jax.ShapeDtypeStruct((NPAD, 1), jnp.float32),
        ],
    )(x, w, a_src, a_dst)


def _tc_mid(acc, haug, wself, b, w, a_src, a_dst):
    return pl.pallas_call(
        _tc_mid_kernel,
        grid=(TC_GRID,),
        in_specs=[
            pl.BlockSpec((NC, TC_BLK, HA), lambda i: (0, i, 0)),
            _row_spec(HA), _row_spec(1), _whole((1, H)),
            _whole((H, H)), _whole((H, 1)), _whole((H, 1)),
        ],
        out_specs=[_row_spec(HA), _row_spec(1), _row_spec(1)],
        out_shape=[
            jax.ShapeDtypeStruct((NPAD, HA), jnp.float32),
            jax.ShapeDtypeStruct((NPAD, 1), jnp.float32),
            jax.ShapeDtypeStruct((NPAD, 1), jnp.float32),
        ],
    )(acc, haug, wself, b, w, a_src, a_dst)


def _tc_fin(acc, haug, wself, b, wl, bl):
    return pl.pallas_call(
        _tc_fin_kernel,
        grid=(TC_GRID,),
        in_specs=[
            pl.BlockSpec((NC, TC_BLK, HA), lambda i: (0, i, 0)),
            _row_spec(HA), _row_spec(1), _whole((1, H)),
            _whole((H, NE)), _whole((1, NE)),
        ],
        out_specs=_row_spec(NE),
        out_shape=jax.ShapeDtypeStruct((NPAD, NE), jnp.float32),
    )(acc, haug, wself, b, wl, bl)


def _sc_gat_body(haug, adv, eidx, out,
                 ebuf, ad_t, w_b, rbuf, acc,
                 gsem0, gsem1, isem0, isem1, ssem):
    c = lax.axis_index("c")
    s = lax.axis_index("s")
    wid = c * NS + s
    gsems = (gsem0, gsem1)
    isems = (isem0, isem1)

    pltpu.sync_copy(adv.at[pl.ds(0, ACC_ROWS)], ad_t)

    zero = jnp.zeros((16,), jnp.float32)

    def zrow(r, carry):
        for q in range(HA // 16):
            rbuf[0, r, pl.ds(q * 16, 16)] = zero
        return carry

    lax.fori_loop(0, CH, zrow, 0)
    base = s * ROWS_PER_TEC
    nfull = ROWS_PER_TEC // CH
    for k in range(nfull):
        pltpu.sync_copy(rbuf.at[0], acc.at[pl.ds(base + k * CH, CH)])
    rem = ROWS_PER_TEC % CH
    if rem:
        pltpu.sync_copy(rbuf.at[0, pl.ds(0, rem)],
                        acc.at[pl.ds(base + nfull * CH, rem)])
    plsc.subcore_barrier()

    pltpu.async_copy(eidx.at[wid, 0], ebuf.at[0], isems[0])
    pltpu.make_async_copy(eidx.at[wid, 0], ebuf.at[0], isems[0]).wait()
    pltpu.async_copy(eidx.at[wid, 1], ebuf.at[1], isems[1])
    pltpu.async_copy(haug.at[ebuf.at[0, 0]], rbuf.at[0], gsems[0])

    def chunk_step(j, par):
        em = j % 4

        @pl.when(j >= 1)
        def _():
            pltpu.make_async_copy(
                rbuf.at[1 - par], acc.at[ebuf.at[(j - 1) % 4, 1]],
                ssem).wait()

        @pl.when(j + 2 < NCH)
        def _():
            pltpu.async_copy(eidx.at[wid, j + 2], ebuf.at[(j + 2) % 4],
                             isems[par])

        @pl.when(j + 1 < NCH)
        def _():
            e1 = (j + 1) % 4
            pltpu.make_async_copy(eidx.at[wid, j + 1], ebuf.at[e1],
                                  isems[1 - par]).wait()
            pltpu.async_copy(haug.at[ebuf.at[e1, 0]], rbuf.at[1 - par],
                             gsems[1 - par])

        pltpu.make_async_copy(haug.at[ebuf.at[em, 0]], rbuf.at[par],
                              gsems[par]).wait()

        for v in range(CH // 16):
            rows16 = lax.iota(jnp.int32, 16) + v * 16
            asg = plsc.load_gather(
                rbuf, [jnp.zeros((16,), jnp.int32) + par, rows16,
                       jnp.full((16,), H + 1, jnp.int32)])
            dv = ebuf[em, 1, pl.ds(v * 16, 16)]
            t = asg + plsc.load_gather(ad_t, [dv])
            w_b[pl.ds(v * 16, 16)] = jnp.exp(jnp.where(t > 0, t, 0.2 * t))

        def srow(r, carry2):
            wv = plsc.load_gather(w_b, [jnp.zeros((16,), jnp.int32) + r])
            for q in range(HA // 16):
                rbuf[par, r, pl.ds(q * 16, 16)] = (
                    rbuf[par, r, pl.ds(q * 16, 16)] * wv)
            return carry2

        lax.fori_loop(0, CH, srow, 0)
        pltpu.async_copy(rbuf.at[par], acc.at[ebuf.at[em, 1]], ssem, add=True)

    def pair_body(t, carry):
        chunk_step(2 * t, 0)
        chunk_step(2 * t + 1, 1)
        return carry

    lax.fori_loop(0, NCH // 2, pair_body, 0)
    pltpu.make_async_copy(rbuf.at[1], acc.at[ebuf.at[(NCH - 1) % 4, 1]],
                          ssem).wait()
    plsc.subcore_barrier()
    pltpu.sync_copy(acc.at[pl.ds(s * ROWS_PER_TEC, ROWS_PER_TEC)],
                    out.at[c, pl.ds(s * ROWS_PER_TEC, ROWS_PER_TEC)])


_sc_gat = functools.partial(
    pl.kernel,
    out_type=jax.ShapeDtypeStruct((NC, NPAD, HA), jnp.float32),
    mesh=plsc.VectorSubcoreMesh(core_axis_name="c", subcore_axis_name="s"),
    compiler_params=pltpu.CompilerParams(
        needs_layout_passes=False, use_tc_tiling_on_sc=False),
    scratch_types=[
        pltpu.VMEM((4, 2, CH), jnp.int32),
        pltpu.VMEM((ACC_ROWS,), jnp.float32),
        pltpu.VMEM((CH,), jnp.float32),
        pltpu.VMEM((2, CH, HA), jnp.float32),
        pltpu.VMEM_SHARED((ACC_ROWS, HA), jnp.float32),
        pltpu.SemaphoreType.DMA,
        pltpu.SemaphoreType.DMA,
        pltpu.SemaphoreType.DMA,
        pltpu.SemaphoreType.DMA,
        pltpu.SemaphoreType.DMA,
    ],
)(_sc_gat_body)


def kernel(x, edge_index, W1, a_src1, a_dst1, b1, W2, a_src2, a_dst2, b2, Wl, bl):
    src = edge_index[0]
    dst = edge_index[1]
    pad = EPAD - E
    srcs = jnp.concatenate(
        [src, jnp.zeros((pad,), jnp.int32)]).reshape(NW, NCH, CH)
    dsts = jnp.concatenate(
        [dst, jnp.full((pad,), JUNK, jnp.int32)]).reshape(NW, NCH, CH)
    eidx = jnp.stack([srcs, dsts], axis=2)

    xp = jnp.concatenate([x, jnp.zeros((NPAD - N, D), jnp.float32)])

    a_src1c = a_src1.reshape(H, 1)
    a_dst1c = a_dst1.reshape(H, 1)
    a_src2c = a_src2.reshape(H, 1)
    a_dst2c = a_dst2.reshape(H, 1)

    haug1, adv1, wself1 = _tc_prep(xp, W1, a_src1c, a_dst1c)
    acc1 = _sc_gat(haug1, adv1.reshape(NPAD), eidx)
    haug2, adv2, wself2 = _tc_mid(
        acc1, haug1, wself1, b1.reshape(1, H), W2, a_src2c, a_dst2c)
    acc2 = _sc_gat(haug2, adv2.reshape(NPAD), eidx)
    out = _tc_fin(acc2, haug2, wself2, b2.reshape(1, H), Wl, bl.reshape(1, NE))

    return out[:N].reshape(-1, 2000 * NE)


# speedup vs baseline: 23.8526x; 1.0924x over previous
import functools

import jax
import jax.numpy as jnp
from jax import lax
from jax.experimental import pallas as pl
from jax.experimental.pallas import tpu as pltpu
from jax.experimental.pallas import tpu_sc as plsc

N = 10000
E = 320000
D = 128
H = 128
NE = 64

NPAD = 10240
HA = 144
NC = 2
NS = 16
NW = NC * NS
CH = 96
NCH_A = 140
NCH_B = 72
LEN_A = NS * NCH_A * CH
LEN_B = NS * NCH_B * CH
EPAD = LEN_A + LEN_B
ACC_ROWS = 10016
JUNK = N + 8
ROWS_PER_TEC = ACC_ROWS // NS
TC_BLK = 512
TC_GRID = NPAD // TC_BLK


def _leaky(x):
    return jnp.where(x > 0, x, 0.2 * x)


def _prep_tail(h, a_src_ref, a_dst_ref, haug_ref, adv_ref, wself_ref):
    asv = jax.lax.dot_general(h, a_src_ref[...], (((1,), (0,)), ((), ())),
                              precision=lax.Precision.HIGHEST,
                              preferred_element_type=jnp.float32)
    adv = jax.lax.dot_general(h, a_dst_ref[...], (((1,), (0,)), ((), ())),
                              precision=lax.Precision.HIGHEST,
                              preferred_element_type=jnp.float32)
    wself = jnp.exp(_leaky(asv + adv))
    tail_iota = lax.broadcasted_iota(jnp.int32, (TC_BLK, HA - H), 1)
    extra = jnp.where(tail_iota == 0, 1.0, jnp.where(tail_iota == 1, asv, 0.0))
    haug_ref[...] = jnp.concatenate([h, extra], axis=1)
    adv_ref[...] = adv
    wself_ref[...] = wself


def _tc_prep_kernel(x_ref, w_ref, a_src_ref, a_dst_ref,
                    haug_ref, adv_ref, wself_ref):
    h = jax.lax.dot_general(x_ref[...], w_ref[...], (((1,), (0,)), ((), ())),
                            precision=lax.Precision.HIGHEST,
                            preferred_element_type=jnp.float32)
    _prep_tail(h, a_src_ref, a_dst_ref, haug_ref, adv_ref, wself_ref)


def _epilogue(acc_ref, haug_ref, wself_ref, b_ref):
    a = acc_ref[0] + acc_ref[1]
    hprev = haug_ref[...][:, :H]
    wself = wself_ref[...]
    numer = a[:, :H] + wself * hprev
    tail_iota = lax.broadcasted_iota(jnp.int32, (TC_BLK, HA - H), 1)
    den = jnp.sum(jnp.where(tail_iota == 0, a[:, H:], 0.0),
                  axis=1, keepdims=True) + wself + 1e-16
    hid = jnp.maximum(numer / den + b_ref[...], 0.0)
    i = pl.program_id(0)
    rows = i * TC_BLK + lax.broadcasted_iota(jnp.int32, (TC_BLK, 1), 0)
    return jnp.where(rows < N, hid, 0.0)


def _tc_mid_kernel(acc_ref, haug_ref, wself_ref, b_ref, w_ref, a_src_ref,
                   a_dst_ref, haug2_ref, adv_ref, wself2_ref):
    hid = _epilogue(acc_ref, haug_ref, wself_ref, b_ref)
    h2 = jax.lax.dot_general(hid, w_ref[...], (((1,), (0,)), ((), ())),
                             precision=lax.Precision.HIGHEST,
                             preferred_element_type=jnp.float32)
    _prep_tail(h2, a_src_ref, a_dst_ref, haug2_ref, adv_ref, wself2_ref)


def _tc_fin_kernel(acc_ref, haug_ref, wself_ref, b_ref, wl_ref, bl_ref, out_ref):
    hid = _epilogue(acc_ref, haug_ref, wself_ref, b_ref)
    out_ref[...] = jax.lax.dot_general(
        hid, wl_ref[...], (((1,), (0,)), ((), ())),
        precision=lax.Precision.HIGHEST,
        preferred_element_type=jnp.float32) + bl_ref[...]


def _row_spec(width):
    return pl.BlockSpec((TC_BLK, width), lambda i: (i, 0))


def _whole(shape):
    return pl.BlockSpec(shape, lambda i: tuple(0 for _ in shape))


def _tc_prep(x, w, a_src, a_dst):
    return pl.pallas_call(
        _tc_prep_kernel,
        grid=(TC_GRID,),
        in_specs=[_row_spec(D), _whole((D, H)), _whole((H, 1)), _whole((H, 1))],
        out_specs=[_row_spec(HA), _row_spec(1), _row_spec(1)],
        out_shape=[
            jax.ShapeDtypeStruct((NPAD, HA), jnp.float32),
            jax.ShapeDtypeStruct((NPAD, 1), jnp.float32),
            jax.ShapeDtypeStruct((NPAD, 1), jnp.float32),
        ],
    )(x, w, a_src, a_dst)


def _tc_mid(acc, haug, wself, b, w, a_src, a_dst):
    return pl.pallas_call(
        _tc_mid_kernel,
        grid=(TC_GRID,),
        in_specs=[
            pl.BlockSpec((NC, TC_BLK, HA), lambda i: (0, i, 0)),
            _row_spec(HA), _row_spec(1), _whole((1, H)),
            _whole((H, H)), _whole((H, 1)), _whole((H, 1)),
        ],
        out_specs=[_row_spec(HA), _row_spec(1), _row_spec(1)],
        out_shape=[
            jax.ShapeDtypeStruct((NPAD, HA), jnp.float32),
            jax.ShapeDtypeStruct((NPAD, 1), jnp.float32),
            jax.ShapeDtypeStruct((NPAD, 1), jnp.float32),
        ],
    )(acc, haug, wself, b, w, a_src, a_dst)


def _tc_fin(acc, haug, wself, b, wl, bl):
    return pl.pallas_call(
        _tc_fin_kernel,
        grid=(TC_GRID,),
        in_specs=[
            pl.BlockSpec((NC, TC_BLK, HA), lambda i: (0, i, 0)),
            _row_spec(HA), _row_spec(1), _whole((1, H)),
            _whole((H, NE)), _whole((1, NE)),
        ],
        out_specs=_row_spec(NE),
        out_shape=jax.ShapeDtypeStruct((NPAD, NE), jnp.float32),
    )(acc, haug, wself, b, wl, bl)


def _sc_gat_body(haug, adv, eidx_a, eidx_b, out,
                 ebuf, ad_t, w_b, rbuf, acc,
                 gsem0, gsem1, isem0, isem1, ssem):
    c = lax.axis_index("c")
    s = lax.axis_index("s")
    gsems = (gsem0, gsem1)
    isems = (isem0, isem1)

    pltpu.sync_copy(adv.at[pl.ds(0, ACC_ROWS)], ad_t)

    zero = jnp.zeros((16,), jnp.float32)

    def zrow(r, carry):
        for q in range(HA // 16):
            rbuf[0, r, pl.ds(q * 16, 16)] = zero
        return carry

    lax.fori_loop(0, CH, zrow, 0)
    base = s * ROWS_PER_TEC
    nfull = ROWS_PER_TEC // CH
    for k in range(nfull):
        pltpu.sync_copy(rbuf.at[0], acc.at[pl.ds(base + k * CH, CH)])
    rem = ROWS_PER_TEC % CH
    if rem:
        pltpu.sync_copy(rbuf.at[0, pl.ds(0, rem)],
                        acc.at[pl.ds(base + nfull * CH, rem)])
    plsc.subcore_barrier()

    def run_core(eidx, nch):
        pltpu.async_copy(eidx.at[s, 0], ebuf.at[0], isems[0])
        pltpu.make_async_copy(eidx.at[s, 0], ebuf.at[0], isems[0]).wait()
        pltpu.async_copy(eidx.at[s, 1], ebuf.at[1], isems[1])
        pltpu.async_copy(haug.at[ebuf.at[0, 0]], rbuf.at[0], gsems[0])

        def chunk_step(j, par):
            em = j % 4

            @pl.when(j >= 1)
            def _():
                pltpu.make_async_copy(
                    rbuf.at[1 - par], acc.at[ebuf.at[(j - 1) % 4, 1]],
                    ssem).wait()

            @pl.when(j + 2 < nch)
            def _():
                pltpu.async_copy(eidx.at[s, j + 2], ebuf.at[(j + 2) % 4],
                                 isems[par])

            @pl.when(j + 1 < nch)
            def _():
                e1 = (j + 1) % 4
                pltpu.make_async_copy(eidx.at[s, j + 1], ebuf.at[e1],
                                      isems[1 - par]).wait()
                pltpu.async_copy(haug.at[ebuf.at[e1, 0]], rbuf.at[1 - par],
                                 gsems[1 - par])

            pltpu.make_async_copy(haug.at[ebuf.at[em, 0]], rbuf.at[par],
                                  gsems[par]).wait()

            for v in range(CH // 16):
                rows16 = lax.iota(jnp.int32, 16) + v * 16
                asg = plsc.load_gather(
                    rbuf, [jnp.zeros((16,), jnp.int32) + par, rows16,
                           jnp.full((16,), H + 1, jnp.int32)])
                dv = ebuf[em, 1, pl.ds(v * 16, 16)]
                t = asg + plsc.load_gather(ad_t, [dv])
                w_b[pl.ds(v * 16, 16)] = jnp.exp(jnp.where(t > 0, t, 0.2 * t))

            def srow(r, carry2):
                wv = plsc.load_gather(w_b, [jnp.zeros((16,), jnp.int32) + r])
                for q in range(HA // 16):
                    rbuf[par, r, pl.ds(q * 16, 16)] = (
                        rbuf[par, r, pl.ds(q * 16, 16)] * wv)
                return carry2

            lax.fori_loop(0, CH, srow, 0)
            pltpu.async_copy(rbuf.at[par], acc.at[ebuf.at[em, 1]], ssem,
                             add=True)

        def pair_body(t, carry):
            chunk_step(2 * t, 0)
            chunk_step(2 * t + 1, 1)
            return carry

        lax.fori_loop(0, nch // 2, pair_body, 0)
        pltpu.make_async_copy(rbuf.at[1], acc.at[ebuf.at[3, 1]],
                              ssem).wait()

    @pl.when(c == 0)
    def _():
        run_core(eidx_a, NCH_A)

    @pl.when(c == 1)
    def _():
        run_core(eidx_b, NCH_B)

    plsc.subcore_barrier()
    pltpu.sync_copy(acc.at[pl.ds(s * ROWS_PER_TEC, ROWS_PER_TEC)],
                    out.at[c, pl.ds(s * ROWS_PER_TEC, ROWS_PER_TEC)])


_sc_gat = functools.partial(
    pl.kernel,
    out_type=jax.ShapeDtypeStruct((NC, NPAD, HA), jnp.float32),
    mesh=plsc.VectorSubcoreMesh(core_axis_name="c", subcore_axis_name="s"),
    compiler_params=pltpu.CompilerParams(
        needs_layout_passes=False, use_tc_tiling_on_sc=False),
    scratch_types=[
        pltpu.VMEM((4, 2, CH), jnp.int32),
        pltpu.VMEM((ACC_ROWS,), jnp.float32),
        pltpu.VMEM((CH,), jnp.float32),
        pltpu.VMEM((2, CH, HA), jnp.float32),
        pltpu.VMEM_SHARED((ACC_ROWS, HA), jnp.float32),
        pltpu.SemaphoreType.DMA,
        pltpu.SemaphoreType.DMA,
        pltpu.SemaphoreType.DMA,
        pltpu.SemaphoreType.DMA,
        pltpu.SemaphoreType.DMA,
    ],
)(_sc_gat_body)


def kernel(x, edge_index, W1, a_src1, a_dst1, b1, W2, a_src2, a_dst2, b2, Wl, bl):
    src = edge_index[0]
    dst = edge_index[1]
    pad = EPAD - E
    srcp = jnp.concatenate([src, jnp.zeros((pad,), jnp.int32)])
    dstp = jnp.concatenate([dst, jnp.full((pad,), JUNK, jnp.int32)])
    eidx_a = jnp.stack([srcp[:LEN_A].reshape(NS, NCH_A, CH),
                        dstp[:LEN_A].reshape(NS, NCH_A, CH)], axis=2)
    eidx_b = jnp.stack([srcp[LEN_A:].reshape(NS, NCH_B, CH),
                        dstp[LEN_A:].reshape(NS, NCH_B, CH)], axis=2)

    xp = jnp.concatenate([x, jnp.zeros((NPAD - N, D), jnp.float32)])

    a_src1c = a_src1.reshape(H, 1)
    a_dst1c = a_dst1.reshape(H, 1)
    a_src2c = a_src2.reshape(H, 1)
    a_dst2c = a_dst2.reshape(H, 1)

    haug1, adv1, wself1 = _tc_prep(xp, W1, a_src1c, a_dst1c)
    acc1 = _sc_gat(haug1, adv1.reshape(NPAD), eidx_a, eidx_b)
    haug2, adv2, wself2 = _tc_mid(
        acc1, haug1, wself1, b1.reshape(1, H), W2, a_src2c, a_dst2c)
    acc2 = _sc_gat(haug2, adv2.reshape(NPAD), eidx_a, eidx_b)
    out = _tc_fin(acc2, haug2, wself2, b2.reshape(1, H), Wl, bl.reshape(1, NE))

    return out[:N].reshape(-1, 2000 * NE)
